# baseline (speedup 1.0000x reference)
import functools

import jax
import jax.numpy as jnp
import numpy as np
from jax import lax
from jax.experimental import pallas as pl
from jax.experimental.pallas import tpu as pltpu
from jax.experimental.pallas import tpu_sc as plsc

_N = 8 * 16 * 196
_F = 128
_HW = 196
_C = 192
_D = 256
_K = 1024
_G = 8

_FPB = 8
_NB = _F // _FPB
_BM = _FPB * _HW

_NC = 2
_NS = 16
_NW = _NC * _NS
_BPW = _N // _NW
_CHUNK = 112
_NCHUNK = _BPW // _CHUNK


def _encode_body(z_ref, w1t_ref, b1_ref, w2t_ref, b2_ref, gamma_ref,
                 beta_ref, cb_ref, idx_ref):
    gs = _D // _G
    cb = cb_ref[...]
    cbn = jnp.sum(cb * cb, axis=1, keepdims=True)
    idx_parts = []
    for f in range(_FPB):
        zf = z_ref[f * _C:(f + 1) * _C, :]
        h = jnp.dot(w1t_ref[...], zf, preferred_element_type=jnp.float32)
        h = jnp.maximum(h + b1_ref[...], 0.0)
        h = jnp.dot(w2t_ref[...], h, preferred_element_type=jnp.float32)
        h = jnp.maximum(h + b2_ref[...], 0.0)
        parts = []
        for g in range(_G):
            seg = h[g * gs:(g + 1) * gs, :]
            m = jnp.mean(seg, axis=0, keepdims=True)
            d = seg - m
            v = jnp.mean(d * d, axis=0, keepdims=True)
            parts.append(d * lax.rsqrt(v + 1e-5))
        hc = jnp.concatenate(parts, axis=0)
        hq = hc * gamma_ref[...] + beta_ref[...]
        hqn = jnp.sum(hq * hq, axis=0, keepdims=True)
        m = jnp.dot(cb, hq, preferred_element_type=jnp.float32)
        d2 = (hqn - 2.0 * m) + cbn
        dmin = jnp.min(d2, axis=0, keepdims=True)
        row = lax.broadcasted_iota(jnp.int32, d2.shape, 0)
        idx_parts.append(jnp.min(jnp.where(d2 == dmin, row, _K), axis=0))
    idx = jnp.concatenate(idx_parts)
    idx_ref[...] = idx.astype(jnp.int32).reshape(1, 1, _BM)


def _decode_body(zq_ref, wout_ref, bout_ref, out_ref):
    yt = lax.dot_general(
        wout_ref[...], zq_ref[...], (((0,), (1,)), ((), ())),
        preferred_element_type=jnp.float32,
    ) + bout_ref[...]
    for f in range(_FPB):
        out_ref[f] = yt[:, f * _HW:(f + 1) * _HW]


def _gather_body(table_hbm, idx_hbm, out_hbm, idx_v, rows_v, sem):
    wid = lax.axis_index("s") * _NC + lax.axis_index("c")
    base = wid * _BPW
    wpr = _BM // _BPW
    pltpu.sync_copy(idx_hbm.at[wid // wpr, 0], idx_v)
    voff = (wid % wpr) * _BPW

    def _gather_chunk(c):
        return pltpu.async_copy(
            table_hbm.at[idx_v.at[pl.ds(voff + c * _CHUNK, _CHUNK)]],
            rows_v.at[c % 2], sem,
        )

    cp = _gather_chunk(0)
    for c in range(_NCHUNK):
        cp.wait()
        if c + 1 < _NCHUNK:
            cp = _gather_chunk(c + 1)
        pltpu.sync_copy(rows_v.at[c % 2],
                        out_hbm.at[pl.ds(base + c * _CHUNK, _CHUNK)])


def kernel(z, W1, b1, W2, b2, gamma, beta, codebook, W_out, b_out):
    B, T = z.shape[0], z.shape[1]
    zt = jnp.swapaxes(z, 2, 3).reshape(_F * _C, _HW)

    idx3 = pl.pallas_call(
        _encode_body,
        grid=(_NB,),
        in_specs=[
            pl.BlockSpec((_FPB * _C, _HW), lambda i: (i, 0)),
            pl.BlockSpec((_D, _C), lambda i: (0, 0)),
            pl.BlockSpec((_D, 1), lambda i: (0, 0)),
            pl.BlockSpec((_D, _D), lambda i: (0, 0)),
            pl.BlockSpec((_D, 1), lambda i: (0, 0)),
            pl.BlockSpec((_D, 1), lambda i: (0, 0)),
            pl.BlockSpec((_D, 1), lambda i: (0, 0)),
            pl.BlockSpec((_K, _D), lambda i: (0, 0)),
        ],
        out_specs=pl.BlockSpec((1, 1, _BM), lambda i: (i, 0, 0)),
        out_shape=jax.ShapeDtypeStruct((_NB, 1, _BM), jnp.int32),
    )(
        zt, W1.T, b1.reshape(_D, 1), W2.T, b2.reshape(_D, 1),
        gamma.reshape(_D, 1), beta.reshape(_D, 1), codebook,
    )

    mesh = plsc.VectorSubcoreMesh(
        core_axis_name="c", subcore_axis_name="s",
        num_cores=_NC, num_subcores=_NS,
    )
    gather = functools.partial(
        pl.kernel,
        out_type=jax.ShapeDtypeStruct((_N, _D), jnp.float32),
        mesh=mesh,
        scratch_types=[
            pltpu.VMEM((_BM,), jnp.int32),
            pltpu.VMEM((2, _CHUNK, _D), jnp.float32),
            pltpu.SemaphoreType.DMA,
        ],
    )(_gather_body)
    zq_flat = gather(codebook, idx3)

    out_t = pl.pallas_call(
        _decode_body,
        grid=(_NB,),
        in_specs=[
            pl.BlockSpec((_BM, _D), lambda i: (i, 0)),
            pl.BlockSpec((_D, _C), lambda i: (0, 0)),
            pl.BlockSpec((_C, 1), lambda i: (0, 0)),
        ],
        out_specs=pl.BlockSpec((_FPB, _C, _HW), lambda i: (i, 0, 0)),
        out_shape=jax.ShapeDtypeStruct((_F, _C, _HW), jnp.float32),
    )(zq_flat, W_out, b_out.reshape(_C, 1))

    return jnp.swapaxes(out_t.reshape(B, T, _C, _HW), 2, 3)


# speedup vs baseline: 1.0264x; 1.0264x over previous
import functools

import jax
import jax.numpy as jnp
import numpy as np
from jax import lax
from jax.experimental import pallas as pl
from jax.experimental.pallas import tpu as pltpu
from jax.experimental.pallas import tpu_sc as plsc

_N = 8 * 16 * 196
_F = 128
_HW = 196
_C = 192
_D = 256
_K = 1024
_G = 8

_FPB = 8
_NB = _F // _FPB
_BM = _FPB * _HW

_NC = 2
_NS = 16
_NW = _NC * _NS
_BPW = _N // _NW
_CHUNK = 392
_NCHUNK = _BPW // _CHUNK


def _encode_body(z_ref, w1t_ref, b1_ref, w2t_ref, b2_ref, gamma_ref,
                 beta_ref, cb_ref, idx_ref):
    gs = _D // _G
    cb = cb_ref[...]
    cbn = jnp.sum(cb * cb, axis=1, keepdims=True)
    idx_parts = []
    for f in range(_FPB):
        zf = z_ref[f * _C:(f + 1) * _C, :]
        h = jnp.dot(w1t_ref[...], zf, preferred_element_type=jnp.float32)
        h = jnp.maximum(h + b1_ref[...], 0.0)
        h = jnp.dot(w2t_ref[...], h, preferred_element_type=jnp.float32)
        h = jnp.maximum(h + b2_ref[...], 0.0)
        parts = []
        for g in range(_G):
            seg = h[g * gs:(g + 1) * gs, :]
            m = jnp.mean(seg, axis=0, keepdims=True)
            d = seg - m
            v = jnp.mean(d * d, axis=0, keepdims=True)
            parts.append(d * lax.rsqrt(v + 1e-5))
        hc = jnp.concatenate(parts, axis=0)
        hq = hc * gamma_ref[...] + beta_ref[...]
        hqn = jnp.sum(hq * hq, axis=0, keepdims=True)
        m = jnp.dot(cb, hq, preferred_element_type=jnp.float32)
        d2 = (hqn - 2.0 * m) + cbn
        dmin = jnp.min(d2, axis=0, keepdims=True)
        row = lax.broadcasted_iota(jnp.int32, d2.shape, 0)
        idx_parts.append(jnp.min(jnp.where(d2 == dmin, row, _K), axis=0))
    idx = jnp.concatenate(idx_parts)
    idx_ref[...] = idx.astype(jnp.int32).reshape(1, 1, _BM)


def _decode_body(zq_ref, wout_ref, bout_ref, out_ref):
    yt = lax.dot_general(
        wout_ref[...], zq_ref[...], (((0,), (1,)), ((), ())),
        preferred_element_type=jnp.float32,
    ) + bout_ref[...]
    for f in range(_FPB):
        out_ref[f] = yt[:, f * _HW:(f + 1) * _HW]


def _gather_body(table_hbm, idx_hbm, out_hbm, idx_v, rows_v, sem):
    wid = lax.axis_index("s") * _NC + lax.axis_index("c")
    base = wid * _BPW
    wpr = _BM // _BPW
    pltpu.sync_copy(idx_hbm.at[wid // wpr, 0], idx_v)
    voff = (wid % wpr) * _BPW

    for c in range(_NCHUNK):
        pltpu.async_copy(
            table_hbm.at[idx_v.at[pl.ds(voff + c * _CHUNK, _CHUNK)]],
            rows_v, sem,
        ).wait()
        pltpu.sync_copy(rows_v,
                        out_hbm.at[pl.ds(base + c * _CHUNK, _CHUNK)])


def kernel(z, W1, b1, W2, b2, gamma, beta, codebook, W_out, b_out):
    B, T = z.shape[0], z.shape[1]
    zt = jnp.swapaxes(z, 2, 3).reshape(_F * _C, _HW)

    idx3 = pl.pallas_call(
        _encode_body,
        grid=(_NB,),
        in_specs=[
            pl.BlockSpec((_FPB * _C, _HW), lambda i: (i, 0)),
            pl.BlockSpec((_D, _C), lambda i: (0, 0)),
            pl.BlockSpec((_D, 1), lambda i: (0, 0)),
            pl.BlockSpec((_D, _D), lambda i: (0, 0)),
            pl.BlockSpec((_D, 1), lambda i: (0, 0)),
            pl.BlockSpec((_D, 1), lambda i: (0, 0)),
            pl.BlockSpec((_D, 1), lambda i: (0, 0)),
            pl.BlockSpec((_K, _D), lambda i: (0, 0)),
        ],
        out_specs=pl.BlockSpec((1, 1, _BM), lambda i: (i, 0, 0)),
        out_shape=jax.ShapeDtypeStruct((_NB, 1, _BM), jnp.int32),
    )(
        zt, W1.T, b1.reshape(_D, 1), W2.T, b2.reshape(_D, 1),
        gamma.reshape(_D, 1), beta.reshape(_D, 1), codebook,
    )

    mesh = plsc.VectorSubcoreMesh(
        core_axis_name="c", subcore_axis_name="s",
        num_cores=_NC, num_subcores=_NS,
    )
    gather = functools.partial(
        pl.kernel,
        out_type=jax.ShapeDtypeStruct((_N, _D), jnp.float32),
        mesh=mesh,
        scratch_types=[
            pltpu.VMEM((_BM,), jnp.int32),
            pltpu.VMEM((_CHUNK, _D), jnp.float32),
            pltpu.SemaphoreType.DMA,
        ],
    )(_gather_body)
    zq_flat = gather(codebook, idx3)

    out_t = pl.pallas_call(
        _decode_body,
        grid=(_NB,),
        in_specs=[
            pl.BlockSpec((_BM, _D), lambda i: (i, 0)),
            pl.BlockSpec((_D, _C), lambda i: (0, 0)),
            pl.BlockSpec((_C, 1), lambda i: (0, 0)),
        ],
        out_specs=pl.BlockSpec((_FPB, _C, _HW), lambda i: (i, 0, 0)),
        out_shape=jax.ShapeDtypeStruct((_F, _C, _HW), jnp.float32),
    )(zq_flat, W_out, b_out.reshape(_C, 1))

    return jnp.swapaxes(out_t.reshape(B, T, _C, _HW), 2, 3)
